# Initial kernel scaffold; baseline (speedup 1.0000x reference)
import jax
import jax.numpy as jnp
from jax.experimental import pallas as pl


def kernel(x, edge_index, W1, b1, W2, b2, W_out, b_out):
    raise NotImplementedError("write your pallas kernel here")


# speedup vs baseline: 9.3482x; 9.3482x over previous
import functools

import jax
import jax.numpy as jnp
from jax import lax
from jax.experimental import pallas as pl
from jax.experimental.pallas import tpu as pltpu
from jax.experimental.pallas import tpu_sc as plsc

N_NODES = 10000
D_FEAT = 128
HIDDEN = 128
N_CLASSES = 64
N_EDGES = 320000

NC = 2
NS = 16
CHUNK = 128
NSTEPS = 80
E_PAD = NC * NS * NSTEPS * CHUNK
ROWS_PAD = 10240
RPT = ROWS_PAD // NS
_MESH = dict(core_axis_name="c", subcore_axis_name="s", num_cores=NC,
             num_subcores=NS)


_EPT = NSTEPS * CHUNK


def _deg_body(dst_hbm, dp_hbm, dst_v, deg_v):
    c = lax.axis_index("c")
    s = lax.axis_index("s")
    wid = c * NS + s
    pltpu.sync_copy(dst_hbm.at[c, s], dst_v)
    zeros16 = jnp.zeros((16,), jnp.float32)

    def zstep(i, carry):
        deg_v[pl.ds(i * 16, 16)] = zeros16
        return carry

    lax.fori_loop(0, ROWS_PAD // 16, zstep, 0)
    ones16 = jnp.ones((16,), jnp.float32)

    def step(t, carry):
        idx = dst_v[pl.ds(t * 16, 16)]
        plsc.addupdate_scatter(deg_v, [idx], ones16)
        return carry

    lax.fori_loop(0, _EPT // 16, step, 0)
    pltpu.sync_copy(deg_v, dp_hbm.at[wid])


_deg_kernel = functools.partial(
    pl.kernel,
    out_type=jax.ShapeDtypeStruct((NC * NS, ROWS_PAD), jnp.float32),
    mesh=plsc.VectorSubcoreMesh(**_MESH),
    compiler_params=pltpu.CompilerParams(needs_layout_passes=False),
    scratch_types=[
        pltpu.VMEM((_EPT,), jnp.int32),
        pltpu.VMEM((ROWS_PAD,), jnp.float32),
    ],
)(_deg_body)


def _prop_body(g_hbm, src_hbm, dst_hbm, zeros_hbm, out_hbm,
               src_v, dst_v, rows_v, sem, acc):
    c = lax.axis_index("c")
    s = lax.axis_index("s")
    for k in range(RPT // CHUNK):
        pltpu.sync_copy(zeros_hbm, acc.at[pl.ds(s * RPT + k * CHUNK, CHUNK)])
    pltpu.sync_copy(src_hbm.at[c, s], src_v)
    pltpu.sync_copy(dst_hbm.at[c, s], dst_v)
    plsc.subcore_barrier()

    def step(j, carry):
        pltpu.async_copy(g_hbm.at[src_v.at[j]], rows_v, sem).wait()
        pltpu.sync_copy(rows_v, acc.at[dst_v.at[j]], add=True)
        return carry

    lax.fori_loop(0, NSTEPS, step, 0)
    plsc.subcore_barrier()
    pltpu.sync_copy(acc.at[pl.ds(s * RPT, RPT)],
                    out_hbm.at[c, pl.ds(s * RPT, RPT)])


_prop_kernel = functools.partial(
    pl.kernel,
    out_type=jax.ShapeDtypeStruct((NC, ROWS_PAD, HIDDEN), jnp.float32),
    mesh=plsc.VectorSubcoreMesh(**_MESH),
    scratch_types=[
        pltpu.VMEM((NSTEPS, CHUNK), jnp.int32),
        pltpu.VMEM((NSTEPS, CHUNK), jnp.int32),
        pltpu.VMEM((CHUNK, HIDDEN), jnp.float32),
        pltpu.SemaphoreType.DMA,
        pltpu.VMEM_SHARED((ROWS_PAD, HIDDEN), jnp.float32),
    ],
)(_prop_body)


_RB = 1280
_GRID = ROWS_PAD // _RB


def _dinv_of(dp_ref):
    deg = jnp.sum(dp_ref[...], axis=0) + 1.0
    return lax.rsqrt(deg)


def _mm_scale_body(x_ref, w_ref, dp_ref, g_ref):
    dinv = _dinv_of(dp_ref)
    g_ref[...] = jnp.dot(x_ref[...], w_ref[...],
                         preferred_element_type=jnp.float32) * dinv[:, None]


def _combine_mm_body(p_ref, g_ref, dp_ref, b_ref, w_ref, o_ref):
    dinv = _dinv_of(dp_ref)
    h = (p_ref[0] + p_ref[1] + g_ref[...]) * dinv[:, None] + b_ref[...]
    h = jnp.maximum(h, 0.0)
    o_ref[...] = jnp.dot(h, w_ref[...],
                         preferred_element_type=jnp.float32) * dinv[:, None]


def _head_body(p_ref, g_ref, dp_ref, b_ref, w_ref, bo_ref, o_ref):
    dinv = _dinv_of(dp_ref)
    h = (p_ref[0] + p_ref[1] + g_ref[...]) * dinv[:, None] + b_ref[...]
    h = jnp.maximum(h, 0.0)
    o_ref[...] = jnp.dot(h, w_ref[...],
                         preferred_element_type=jnp.float32) + bo_ref[...]


def _row_spec(width):
    return pl.BlockSpec((_RB, width), lambda i: (i, 0))


_DP_SPEC = pl.BlockSpec((NC * NS, _RB), lambda i: (0, i))
_P_SPEC = pl.BlockSpec((NC, _RB, HIDDEN), lambda i: (0, i, 0))


def _full(shape):
    return pl.BlockSpec(shape, lambda i: tuple(0 for _ in shape))


_mm_scale = pl.pallas_call(
    _mm_scale_body,
    grid=(_GRID,),
    in_specs=[_row_spec(D_FEAT), _full((D_FEAT, HIDDEN)), _DP_SPEC],
    out_specs=_row_spec(HIDDEN),
    out_shape=jax.ShapeDtypeStruct((ROWS_PAD, HIDDEN), jnp.float32),
)

_combine_mm = pl.pallas_call(
    _combine_mm_body,
    grid=(_GRID,),
    in_specs=[_P_SPEC, _row_spec(HIDDEN), _DP_SPEC, _full((1, HIDDEN)),
              _full((HIDDEN, HIDDEN))],
    out_specs=_row_spec(HIDDEN),
    out_shape=jax.ShapeDtypeStruct((ROWS_PAD, HIDDEN), jnp.float32),
)

_head = pl.pallas_call(
    _head_body,
    grid=(_GRID,),
    in_specs=[_P_SPEC, _row_spec(HIDDEN), _DP_SPEC, _full((1, HIDDEN)),
              _full((HIDDEN, N_CLASSES)), _full((1, N_CLASSES))],
    out_specs=_row_spec(N_CLASSES),
    out_shape=jax.ShapeDtypeStruct((ROWS_PAD, N_CLASSES), jnp.float32),
)


def kernel(x, edge_index, W1, b1, W2, b2, W_out, b_out):
    ei = edge_index.astype(jnp.int32)
    pad = E_PAD - N_EDGES
    src = jnp.concatenate([ei[0], jnp.zeros((pad,), jnp.int32)])
    dst = jnp.concatenate([ei[1], jnp.full((pad,), N_NODES, jnp.int32)])
    src4 = src.reshape(NC, NS, NSTEPS, CHUNK)
    dst4 = dst.reshape(NC, NS, NSTEPS, CHUNK)

    zeros128 = jnp.zeros((CHUNK, HIDDEN), jnp.float32)

    dp = _deg_kernel(dst.reshape(NC, NS, _EPT))

    x_p = jnp.pad(x, ((0, ROWS_PAD - N_NODES), (0, 0)))
    g1 = _mm_scale(x_p, W1, dp)
    p1 = _prop_kernel(g1, src4, dst4, zeros128)
    g2 = _combine_mm(p1, g1, dp, b1.reshape(1, HIDDEN), W2)
    p2 = _prop_kernel(g2, src4, dst4, zeros128)
    out = _head(p2, g2, dp, b2.reshape(1, HIDDEN), W_out,
                b_out.reshape(1, N_CLASSES))
    return out[:N_NODES]


# speedup vs baseline: 10.3140x; 1.1033x over previous
import functools

import jax
import jax.numpy as jnp
from jax import lax
from jax.experimental import pallas as pl
from jax.experimental.pallas import tpu as pltpu
from jax.experimental.pallas import tpu_sc as plsc

N_NODES = 10000
D_FEAT = 128
HIDDEN = 128
N_CLASSES = 64
N_EDGES = 320000

NC = 2
NS = 16
CHUNK = 128
NSTEPS = 80
E_PAD = NC * NS * NSTEPS * CHUNK
ROWS_PAD = 10240
RPT = ROWS_PAD // NS
_MESH = dict(core_axis_name="c", subcore_axis_name="s", num_cores=NC,
             num_subcores=NS)


_EPT = NSTEPS * CHUNK


def _deg_body(dst_hbm, dp_hbm, dst_v, deg_v):
    c = lax.axis_index("c")
    s = lax.axis_index("s")
    wid = c * NS + s
    pltpu.sync_copy(dst_hbm.at[c, s], dst_v)
    zeros16 = jnp.zeros((16,), jnp.float32)

    def zstep(i, carry):
        deg_v[pl.ds(i * 16, 16)] = zeros16
        return carry

    lax.fori_loop(0, ROWS_PAD // 16, zstep, 0)
    ones16 = jnp.ones((16,), jnp.float32)

    def step(t, carry):
        idx = dst_v[pl.ds(t * 16, 16)]
        plsc.addupdate_scatter(deg_v, [idx], ones16)
        return carry

    lax.fori_loop(0, _EPT // 16, step, 0)
    pltpu.sync_copy(deg_v, dp_hbm.at[wid])


_deg_kernel = functools.partial(
    pl.kernel,
    out_type=jax.ShapeDtypeStruct((NC * NS, ROWS_PAD), jnp.float32),
    mesh=plsc.VectorSubcoreMesh(**_MESH),
    compiler_params=pltpu.CompilerParams(needs_layout_passes=False),
    scratch_types=[
        pltpu.VMEM((_EPT,), jnp.int32),
        pltpu.VMEM((ROWS_PAD,), jnp.float32),
    ],
)(_deg_body)


def _prop_body(g_hbm, e_hbm, zeros_hbm, out_hbm,
               ia, ib, rows_a, rows_b, sem_ia, sem_ib, sem_a, sem_b, acc):
    c = lax.axis_index("c")
    s = lax.axis_index("s")
    for k in range(RPT // CHUNK):
        pltpu.sync_copy(zeros_hbm, acc.at[pl.ds(s * RPT + k * CHUNK, CHUNK)])

    pltpu.async_copy(e_hbm.at[c, s, 0], ia, sem_ia)
    pltpu.async_copy(e_hbm.at[c, s, 1], ib, sem_ib)
    pltpu.make_async_copy(e_hbm.at[c, s, 0], ia, sem_ia).wait()
    pltpu.async_copy(g_hbm.at[ia.at[0]], rows_a, sem_a)
    pltpu.make_async_copy(e_hbm.at[c, s, 1], ib, sem_ib).wait()
    pltpu.async_copy(g_hbm.at[ib.at[0]], rows_b, sem_b)
    plsc.subcore_barrier()

    def step(i, carry):
        t = 2 * i
        pltpu.make_async_copy(g_hbm.at[ia.at[0]], rows_a, sem_a).wait()
        pltpu.sync_copy(rows_a, acc.at[ia.at[1]], add=True)
        pltpu.async_copy(e_hbm.at[c, s, t + 2], ia, sem_ia)
        pltpu.make_async_copy(e_hbm.at[c, s, t + 2], ia, sem_ia).wait()
        pltpu.async_copy(g_hbm.at[ia.at[0]], rows_a, sem_a)
        pltpu.make_async_copy(g_hbm.at[ib.at[0]], rows_b, sem_b).wait()
        pltpu.sync_copy(rows_b, acc.at[ib.at[1]], add=True)
        pltpu.async_copy(e_hbm.at[c, s, t + 3], ib, sem_ib)
        pltpu.make_async_copy(e_hbm.at[c, s, t + 3], ib, sem_ib).wait()
        pltpu.async_copy(g_hbm.at[ib.at[0]], rows_b, sem_b)
        return carry

    lax.fori_loop(0, NSTEPS // 2 - 1, step, 0)
    pltpu.make_async_copy(g_hbm.at[ia.at[0]], rows_a, sem_a).wait()
    pltpu.sync_copy(rows_a, acc.at[ia.at[1]], add=True)
    pltpu.make_async_copy(g_hbm.at[ib.at[0]], rows_b, sem_b).wait()
    pltpu.sync_copy(rows_b, acc.at[ib.at[1]], add=True)

    plsc.subcore_barrier()
    pltpu.sync_copy(acc.at[pl.ds(s * RPT, RPT)],
                    out_hbm.at[c, pl.ds(s * RPT, RPT)])


_prop_kernel = functools.partial(
    pl.kernel,
    out_type=jax.ShapeDtypeStruct((NC, ROWS_PAD, HIDDEN), jnp.float32),
    mesh=plsc.VectorSubcoreMesh(**_MESH),
    scratch_types=[
        pltpu.VMEM((2, CHUNK), jnp.int32),
        pltpu.VMEM((2, CHUNK), jnp.int32),
        pltpu.VMEM((CHUNK, HIDDEN), jnp.float32),
        pltpu.VMEM((CHUNK, HIDDEN), jnp.float32),
        pltpu.SemaphoreType.DMA,
        pltpu.SemaphoreType.DMA,
        pltpu.SemaphoreType.DMA,
        pltpu.SemaphoreType.DMA,
        pltpu.VMEM_SHARED((ROWS_PAD, HIDDEN), jnp.float32),
    ],
)(_prop_body)


_RB = 1280
_GRID = ROWS_PAD // _RB


def _dinv_of(dp_ref):
    deg = jnp.sum(dp_ref[...], axis=0) + 1.0
    return lax.rsqrt(deg)


def _mm_scale_body(x_ref, w_ref, dp_ref, g_ref):
    dinv = _dinv_of(dp_ref)
    g_ref[...] = jnp.dot(x_ref[...], w_ref[...],
                         preferred_element_type=jnp.float32) * dinv[:, None]


def _combine_mm_body(p_ref, g_ref, dp_ref, b_ref, w_ref, o_ref):
    dinv = _dinv_of(dp_ref)
    h = (p_ref[0] + p_ref[1] + g_ref[...]) * dinv[:, None] + b_ref[...]
    h = jnp.maximum(h, 0.0)
    o_ref[...] = jnp.dot(h, w_ref[...],
                         preferred_element_type=jnp.float32) * dinv[:, None]


def _head_body(p_ref, g_ref, dp_ref, b_ref, w_ref, bo_ref, o_ref):
    dinv = _dinv_of(dp_ref)
    h = (p_ref[0] + p_ref[1] + g_ref[...]) * dinv[:, None] + b_ref[...]
    h = jnp.maximum(h, 0.0)
    o_ref[...] = jnp.dot(h, w_ref[...],
                         preferred_element_type=jnp.float32) + bo_ref[...]


def _row_spec(width):
    return pl.BlockSpec((_RB, width), lambda i: (i, 0))


_DP_SPEC = pl.BlockSpec((NC * NS, _RB), lambda i: (0, i))
_P_SPEC = pl.BlockSpec((NC, _RB, HIDDEN), lambda i: (0, i, 0))


def _full(shape):
    return pl.BlockSpec(shape, lambda i: tuple(0 for _ in shape))


_mm_scale = pl.pallas_call(
    _mm_scale_body,
    grid=(_GRID,),
    in_specs=[_row_spec(D_FEAT), _full((D_FEAT, HIDDEN)), _DP_SPEC],
    out_specs=_row_spec(HIDDEN),
    out_shape=jax.ShapeDtypeStruct((ROWS_PAD, HIDDEN), jnp.float32),
)

_combine_mm = pl.pallas_call(
    _combine_mm_body,
    grid=(_GRID,),
    in_specs=[_P_SPEC, _row_spec(HIDDEN), _DP_SPEC, _full((1, HIDDEN)),
              _full((HIDDEN, HIDDEN))],
    out_specs=_row_spec(HIDDEN),
    out_shape=jax.ShapeDtypeStruct((ROWS_PAD, HIDDEN), jnp.float32),
)

_head = pl.pallas_call(
    _head_body,
    grid=(_GRID,),
    in_specs=[_P_SPEC, _row_spec(HIDDEN), _DP_SPEC, _full((1, HIDDEN)),
              _full((HIDDEN, N_CLASSES)), _full((1, N_CLASSES))],
    out_specs=_row_spec(N_CLASSES),
    out_shape=jax.ShapeDtypeStruct((ROWS_PAD, N_CLASSES), jnp.float32),
)


def kernel(x, edge_index, W1, b1, W2, b2, W_out, b_out):
    ei = edge_index.astype(jnp.int32)
    pad = E_PAD - N_EDGES
    src = jnp.concatenate([ei[0], jnp.zeros((pad,), jnp.int32)])
    dst = jnp.concatenate([ei[1], jnp.full((pad,), N_NODES, jnp.int32)])
    e5 = jnp.stack([src.reshape(NC, NS, NSTEPS, CHUNK),
                    dst.reshape(NC, NS, NSTEPS, CHUNK)], axis=3)

    zeros128 = jnp.zeros((CHUNK, HIDDEN), jnp.float32)

    dp = _deg_kernel(dst.reshape(NC, NS, _EPT))

    x_p = jnp.pad(x, ((0, ROWS_PAD - N_NODES), (0, 0)))
    g1 = _mm_scale(x_p, W1, dp)
    p1 = _prop_kernel(g1, e5, zeros128)
    g2 = _combine_mm(p1, g1, dp, b1.reshape(1, HIDDEN), W2)
    p2 = _prop_kernel(g2, e5, zeros128)
    out = _head(p2, g2, dp, b2.reshape(1, HIDDEN), W_out,
                b_out.reshape(1, N_CLASSES))
    return out[:N_NODES]


# speedup vs baseline: 10.3798x; 1.0064x over previous
import functools

import jax
import jax.numpy as jnp
from jax import lax
from jax.experimental import pallas as pl
from jax.experimental.pallas import tpu as pltpu
from jax.experimental.pallas import tpu_sc as plsc

N_NODES = 10000
D_FEAT = 128
HIDDEN = 128
N_CLASSES = 64
N_EDGES = 320000

NC = 2
NS = 16
CHUNK = 128
NSTEPS = 80
E_PAD = NC * NS * NSTEPS * CHUNK
ROWS_PAD = 10240
RPT = ROWS_PAD // NS
_MESH = dict(core_axis_name="c", subcore_axis_name="s", num_cores=NC,
             num_subcores=NS)


_EPT = NSTEPS * CHUNK


def _deg_body(dst_hbm, dp_hbm, dst_v, deg_v):
    c = lax.axis_index("c")
    s = lax.axis_index("s")
    wid = c * NS + s
    pltpu.sync_copy(dst_hbm.at[c, s], dst_v)
    zeros16 = jnp.zeros((16,), jnp.float32)

    def zstep(i, carry):
        deg_v[pl.ds(i * 16, 16)] = zeros16
        return carry

    lax.fori_loop(0, ROWS_PAD // 16, zstep, 0)
    ones16 = jnp.ones((16,), jnp.float32)

    def step(t, carry):
        idx = dst_v[pl.ds(t * 16, 16)]
        plsc.addupdate_scatter(deg_v, [idx], ones16)
        return carry

    lax.fori_loop(0, _EPT // 16, step, 0)
    pltpu.sync_copy(deg_v, dp_hbm.at[wid])


_deg_kernel = functools.partial(
    pl.kernel,
    out_type=jax.ShapeDtypeStruct((NC * NS, ROWS_PAD), jnp.float32),
    mesh=plsc.VectorSubcoreMesh(**_MESH),
    compiler_params=pltpu.CompilerParams(needs_layout_passes=False),
    scratch_types=[
        pltpu.VMEM((_EPT,), jnp.int32),
        pltpu.VMEM((ROWS_PAD,), jnp.float32),
    ],
)(_deg_body)


def _prop_body(g_hbm, e_hbm, zeros_hbm, out_hbm,
               ia, ib, rows_a, rows_b, sem_ia, sem_ib, sem_a, sem_b, acc):
    c = lax.axis_index("c")
    s = lax.axis_index("s")
    for k in range(RPT // CHUNK):
        pltpu.sync_copy(zeros_hbm, acc.at[pl.ds(s * RPT + k * CHUNK, CHUNK)])

    pltpu.async_copy(e_hbm.at[c, s, 0], ia, sem_ia)
    pltpu.async_copy(e_hbm.at[c, s, 1], ib, sem_ib)
    pltpu.make_async_copy(e_hbm.at[c, s, 0], ia, sem_ia).wait()
    pltpu.async_copy(g_hbm.at[ia.at[0]], rows_a, sem_a)
    pltpu.make_async_copy(e_hbm.at[c, s, 1], ib, sem_ib).wait()
    pltpu.async_copy(g_hbm.at[ib.at[0]], rows_b, sem_b)
    plsc.subcore_barrier()

    def step(i, carry):
        t = 2 * i
        pltpu.make_async_copy(g_hbm.at[ia.at[0]], rows_a, sem_a).wait()
        pltpu.sync_copy(rows_a, acc.at[ia.at[1]], add=True)
        pltpu.async_copy(e_hbm.at[c, s, t + 2], ia, sem_ia)
        pltpu.make_async_copy(e_hbm.at[c, s, t + 2], ia, sem_ia).wait()
        pltpu.async_copy(g_hbm.at[ia.at[0]], rows_a, sem_a)
        pltpu.make_async_copy(g_hbm.at[ib.at[0]], rows_b, sem_b).wait()
        pltpu.sync_copy(rows_b, acc.at[ib.at[1]], add=True)
        pltpu.async_copy(e_hbm.at[c, s, t + 3], ib, sem_ib)
        pltpu.make_async_copy(e_hbm.at[c, s, t + 3], ib, sem_ib).wait()
        pltpu.async_copy(g_hbm.at[ib.at[0]], rows_b, sem_b)
        return carry

    lax.fori_loop(0, NSTEPS // 2 - 1, step, 0)
    pltpu.make_async_copy(g_hbm.at[ia.at[0]], rows_a, sem_a).wait()
    pltpu.sync_copy(rows_a, acc.at[ia.at[1]], add=True)
    pltpu.make_async_copy(g_hbm.at[ib.at[0]], rows_b, sem_b).wait()
    pltpu.sync_copy(rows_b, acc.at[ib.at[1]], add=True)

    plsc.subcore_barrier()
    pltpu.sync_copy(acc.at[pl.ds(s * RPT, RPT)],
                    out_hbm.at[c, pl.ds(s * RPT, RPT)])


_prop_kernel = functools.partial(
    pl.kernel,
    out_type=jax.ShapeDtypeStruct((NC, ROWS_PAD, HIDDEN), jnp.float32),
    mesh=plsc.VectorSubcoreMesh(**_MESH),
    scratch_types=[
        pltpu.VMEM((2, CHUNK), jnp.int32),
        pltpu.VMEM((2, CHUNK), jnp.int32),
        pltpu.VMEM((CHUNK, HIDDEN), jnp.float32),
        pltpu.VMEM((CHUNK, HIDDEN), jnp.float32),
        pltpu.SemaphoreType.DMA,
        pltpu.SemaphoreType.DMA,
        pltpu.SemaphoreType.DMA,
        pltpu.SemaphoreType.DMA,
        pltpu.VMEM_SHARED((ROWS_PAD, HIDDEN), jnp.float32),
    ],
)(_prop_body)


_RB = 1280
_GRID = ROWS_PAD // _RB


def _dinv_of(dp_ref):
    deg = jnp.sum(dp_ref[...], axis=0) + 1.0
    return lax.rsqrt(deg)


def _mm_scale_body(x_ref, w_ref, dp_ref, g_ref):
    dinv = _dinv_of(dp_ref)
    g_ref[...] = jnp.dot(x_ref[...], w_ref[...],
                         preferred_element_type=jnp.float32) * dinv[:, None]


def _combine_mm_body(p_ref, g_ref, dp_ref, b_ref, w_ref, o_ref):
    dinv = _dinv_of(dp_ref)
    h = (p_ref[0] + p_ref[1] + g_ref[...]) * dinv[:, None] + b_ref[...]
    h = jnp.maximum(h, 0.0)
    o_ref[...] = jnp.dot(h, w_ref[...],
                         preferred_element_type=jnp.float32) * dinv[:, None]


def _head_body(p_ref, g_ref, dp_ref, b_ref, w_ref, bo_ref, o_ref):
    dinv = _dinv_of(dp_ref)
    h = (p_ref[0] + p_ref[1] + g_ref[...]) * dinv[:, None] + b_ref[...]
    h = jnp.maximum(h, 0.0)
    o_ref[...] = jnp.dot(h, w_ref[...],
                         preferred_element_type=jnp.float32) + bo_ref[...]


def _row_spec(width):
    return pl.BlockSpec((_RB, width), lambda i: (i, 0))


_DP_SPEC = pl.BlockSpec((NC * NS, _RB), lambda i: (0, i))
_P_SPEC = pl.BlockSpec((NC, _RB, HIDDEN), lambda i: (0, i, 0))


def _full(shape):
    return pl.BlockSpec(shape, lambda i: tuple(0 for _ in shape))


_mm_scale = pl.pallas_call(
    _mm_scale_body,
    grid=(_GRID,),
    in_specs=[_row_spec(D_FEAT), _full((D_FEAT, HIDDEN)), _DP_SPEC],
    out_specs=_row_spec(HIDDEN),
    out_shape=jax.ShapeDtypeStruct((ROWS_PAD, HIDDEN), jnp.float32),
)

_combine_mm = pl.pallas_call(
    _combine_mm_body,
    grid=(_GRID,),
    in_specs=[_P_SPEC, _row_spec(HIDDEN), _DP_SPEC, _full((1, HIDDEN)),
              _full((HIDDEN, HIDDEN))],
    out_specs=_row_spec(HIDDEN),
    out_shape=jax.ShapeDtypeStruct((ROWS_PAD, HIDDEN), jnp.float32),
)

_head = pl.pallas_call(
    _head_body,
    grid=(_GRID,),
    in_specs=[_P_SPEC, _row_spec(HIDDEN), _DP_SPEC, _full((1, HIDDEN)),
              _full((HIDDEN, N_CLASSES)), _full((1, N_CLASSES))],
    out_specs=_row_spec(N_CLASSES),
    out_shape=jax.ShapeDtypeStruct((ROWS_PAD, N_CLASSES), jnp.float32),
)


def kernel(x, edge_index, W1, b1, W2, b2, W_out, b_out):
    ei = edge_index.astype(jnp.int32)
    pad = E_PAD - N_EDGES
    src = jnp.concatenate([ei[0], jnp.zeros((pad,), jnp.int32)])
    pad_dst = N_NODES + (jnp.arange(pad, dtype=jnp.int32)
                         % (ROWS_PAD - N_NODES))
    dst = jnp.concatenate([ei[1], pad_dst])
    e5 = jnp.stack([src.reshape(NC, NS, NSTEPS, CHUNK),
                    dst.reshape(NC, NS, NSTEPS, CHUNK)], axis=3)

    zeros128 = jnp.zeros((CHUNK, HIDDEN), jnp.float32)

    dp = _deg_kernel(dst.reshape(NC, NS, _EPT))

    x_p = jnp.pad(x, ((0, ROWS_PAD - N_NODES), (0, 0)))
    g1 = _mm_scale(x_p, W1, dp)
    p1 = _prop_kernel(g1, e5, zeros128)
    g2 = _combine_mm(p1, g1, dp, b1.reshape(1, HIDDEN), W2)
    p2 = _prop_kernel(g2, e5, zeros128)
    out = _head(p2, g2, dp, b2.reshape(1, HIDDEN), W_out,
                b_out.reshape(1, N_CLASSES))
    return out[:N_NODES]
